# baseline (speedup 1.0000x reference)
import functools

import jax
import jax.numpy as jnp
from jax import lax
from jax.experimental import pallas as pl
from jax.experimental.pallas import tpu as pltpu
from jax.experimental.pallas import tpu_sc as plsc

NC = 2
NS = 16
NW = NC * NS
B = 4096
T = 20
J = 7
D = 64
IDX_PER_B = T * J
B_PER_W = B // NW
CHUNK = 128
N_CHUNKS = B_PER_W * IDX_PER_B // CHUNK
SEGS_PER_W = 2 * B_PER_W
NBUF = 2


def _sc_pooled_lookup(table, idx_blocks, seg_blocks):
    mesh = plsc.VectorSubcoreMesh(core_axis_name="c", subcore_axis_name="s")

    @functools.partial(
        pl.kernel,
        out_type=jax.ShapeDtypeStruct((NW * SEGS_PER_W, D), jnp.float32),
        mesh=mesh,
        compiler_params=pltpu.CompilerParams(use_tc_tiling_on_sc=False),
        scratch_types=[
            pltpu.VMEM((N_CHUNKS * CHUNK,), jnp.int32),
            pltpu.VMEM((N_CHUNKS, CHUNK), jnp.int32),
            pltpu.VMEM_SHARED((NS * SEGS_PER_W, D), jnp.float32),
            pltpu.VMEM((CHUNK, D), jnp.float32),
            pltpu.SemaphoreType.DMA,
        ],
    )
    def k(table_hbm, idx_hbm, seg_hbm, out_hbm, idx_v, seg_v, acc, rows_buf,
          gsem):
        rows = [rows_buf]
        cid = lax.axis_index("c")
        sid = lax.axis_index("s")
        wid = sid * NC + cid
        pltpu.sync_copy(idx_hbm.at[wid], idx_v)
        pltpu.sync_copy(seg_hbm.at[sid], seg_v)

        zeros = jnp.zeros((16,), jnp.float32)

        @pl.loop(0, CHUNK)
        def _(i):
            @pl.loop(0, D, step=16)
            def _(j):
                rows[0][i, pl.ds(j, 16)] = zeros

        @pl.loop(0, SEGS_PER_W, step=CHUNK)
        def _(i):
            pltpu.sync_copy(
                rows[0].at[pl.ds(0, CHUNK)],
                acc.at[pl.ds(sid * SEGS_PER_W + i, CHUNK)],
            )

        @pl.loop(0, N_CHUNKS)
        def _(c):
            gathers = []
            for k in range(CHUNK // 16):
                iv = idx_v[pl.ds(c * CHUNK + k * 16, 16)]
                gathers.append(
                    pltpu.async_copy(
                        table_hbm.at[iv],
                        rows[0].at[pl.ds(k * 16, 16)],
                        gsem,
                    )
                )
            for g in gathers:
                g.wait()
            pltpu.sync_copy(rows[0], acc.at[seg_v.at[c]], add=True)

        pltpu.sync_copy(
            acc.at[pl.ds(sid * SEGS_PER_W, SEGS_PER_W)],
            out_hbm.at[pl.ds(wid * SEGS_PER_W, SEGS_PER_W)],
        )

    return k(table, idx_blocks, seg_blocks)


def kernel(input, table):
    idx = jnp.transpose(input.astype(jnp.int32), (0, 2, 1)).reshape(B, IDX_PER_B)
    idx_blocks = idx.reshape(NW, N_CHUNKS * CHUNK)

    q = jnp.arange(B_PER_W * IDX_PER_B, dtype=jnp.int32)
    seg = 2 * (q // IDX_PER_B) + (q % IDX_PER_B >= T).astype(jnp.int32)
    seg_blocks = (
        seg[None, :] + (jnp.arange(NS, dtype=jnp.int32) * SEGS_PER_W)[:, None]
    ).reshape(NS, N_CHUNKS, CHUNK)

    out = _sc_pooled_lookup(table, idx_blocks, seg_blocks)
    return out.reshape(B, 2 * D)


# speedup vs baseline: 1.0694x; 1.0694x over previous
import functools

import jax
import jax.numpy as jnp
from jax import lax
from jax.experimental import pallas as pl
from jax.experimental.pallas import tpu as pltpu
from jax.experimental.pallas import tpu_sc as plsc

NC = 2
NS = 16
NW = NC * NS
B = 4096
T = 20
J = 7
D = 64
IDX_PER_B = T * J
B_PER_W = B // NW
CHUNK = 128
N_CHUNKS = B_PER_W * IDX_PER_B // CHUNK
SEGS_PER_W = 2 * B_PER_W
NBUF = 2


def _sc_pooled_lookup(table, idx_blocks, seg_blocks):
    mesh = plsc.VectorSubcoreMesh(core_axis_name="c", subcore_axis_name="s")

    @functools.partial(
        pl.kernel,
        out_type=jax.ShapeDtypeStruct((NW * SEGS_PER_W, D), jnp.float32),
        mesh=mesh,
        compiler_params=pltpu.CompilerParams(use_tc_tiling_on_sc=False),
        scratch_types=[
            pltpu.VMEM((N_CHUNKS * CHUNK,), jnp.int32),
            pltpu.VMEM((N_CHUNKS, CHUNK), jnp.int32),
            pltpu.VMEM_SHARED((NS * SEGS_PER_W, D), jnp.float32),
            pltpu.VMEM((CHUNK, D), jnp.float32),
            pltpu.VMEM((CHUNK, D), jnp.float32),
            pltpu.SemaphoreType.DMA,
            pltpu.SemaphoreType.DMA,
        ],
    )
    def k(table_hbm, idx_hbm, seg_hbm, out_hbm, idx_v, seg_v, acc, rows_a,
          rows_b, gsem_a, gsem_b):
        rows = [rows_a, rows_b]
        gsems = [gsem_a, gsem_b]
        cid = lax.axis_index("c")
        sid = lax.axis_index("s")
        wid = sid * NC + cid
        pltpu.sync_copy(idx_hbm.at[wid], idx_v)
        pltpu.sync_copy(seg_hbm.at[sid], seg_v)

        zeros = jnp.zeros((16,), jnp.float32)

        @pl.loop(0, CHUNK)
        def _(i):
            @pl.loop(0, D, step=16)
            def _(j):
                rows[0][i, pl.ds(j, 16)] = zeros

        @pl.loop(0, SEGS_PER_W, step=CHUNK)
        def _(i):
            pltpu.sync_copy(
                rows[0].at[pl.ds(0, CHUNK)],
                acc.at[pl.ds(sid * SEGS_PER_W + i, CHUNK)],
            )

        def fire(c, j):
            return pltpu.async_copy(
                table_hbm.at[idx_v.at[pl.ds(c * CHUNK, CHUNK)]],
                rows[j],
                gsems[j],
            )

        def drain_scatter(c, j):
            pltpu.make_async_copy(
                table_hbm.at[idx_v.at[pl.ds(c * CHUNK, CHUNK)]],
                rows[j],
                gsems[j],
            ).wait()
            pltpu.sync_copy(rows[j], acc.at[seg_v.at[c]], add=True)

        fire(0, 0)

        @pl.loop(0, N_CHUNKS // 2)
        def _(i):
            c = 2 * i
            fire(c + 1, 1)
            drain_scatter(c, 0)

            @pl.when(c + 2 < N_CHUNKS)
            def _():
                fire(c + 2, 0)

            drain_scatter(c + 1, 1)

        pltpu.sync_copy(
            acc.at[pl.ds(sid * SEGS_PER_W, SEGS_PER_W)],
            out_hbm.at[pl.ds(wid * SEGS_PER_W, SEGS_PER_W)],
        )

    return k(table, idx_blocks, seg_blocks)


def kernel(input, table):
    idx = jnp.transpose(input.astype(jnp.int32), (0, 2, 1)).reshape(B, IDX_PER_B)
    idx_blocks = idx.reshape(NW, N_CHUNKS * CHUNK)

    q = jnp.arange(B_PER_W * IDX_PER_B, dtype=jnp.int32)
    seg = 2 * (q // IDX_PER_B) + (q % IDX_PER_B >= T).astype(jnp.int32)
    seg_blocks = (
        seg[None, :] + (jnp.arange(NS, dtype=jnp.int32) * SEGS_PER_W)[:, None]
    ).reshape(NS, N_CHUNKS, CHUNK)

    out = _sc_pooled_lookup(table, idx_blocks, seg_blocks)
    return out.reshape(B, 2 * D)
